# baseline (speedup 1.0000x reference)
import functools

import jax
import jax.numpy as jnp
from jax import lax
from jax.experimental import pallas as pl
from jax.experimental.pallas import tpu as pltpu
from jax.experimental.pallas import tpu_sc as plsc

_ROWS = 8192
_DIM = 4096
_L = 16
_NC = 2
_NS = 16
_NW = _NC * _NS
_RPT = _ROWS // _NW
_RB = 8
_NBLK = _RPT // _RB
_JC = _DIM // _L

_mesh = plsc.VectorSubcoreMesh(core_axis_name="c", subcore_axis_name="s")


@functools.partial(
    pl.kernel,
    out_type=jax.ShapeDtypeStruct((_ROWS * _DIM,), jnp.float32),
    mesh=_mesh,
    scratch_types=[
        pltpu.VMEM((_DIM,), jnp.int32),
        pltpu.VMEM((_RB * _DIM,), jnp.float32),
        pltpu.VMEM((_RB * _DIM,), jnp.float32),
    ],
    compiler_params=pltpu.CompilerParams(needs_layout_passes=False),
)
def _permute_cols(x_hbm, perm_hbm, out_hbm, perm_v, in_v, out_v):
    wid = lax.axis_index("s") * _NC + lax.axis_index("c")
    tile_base = wid * _RPT * _DIM
    pltpu.sync_copy(perm_hbm, perm_v)

    def block_body(b, carry):
        base = tile_base + b * (_RB * _DIM)
        pltpu.sync_copy(x_hbm.at[pl.ds(base, _RB * _DIM)], in_v)

        def col_body(jc, carry):
            idx = perm_v[pl.ds(jc * _L, _L)]
            for r in range(_RB):
                vals = plsc.load_gather(in_v, [idx + r * _DIM])
                out_v[pl.ds(r * _DIM + jc * _L, _L)] = vals
            return carry

        lax.fori_loop(0, _JC, col_body, 0)
        pltpu.sync_copy(out_v, out_hbm.at[pl.ds(base, _RB * _DIM)])
        return carry

    lax.fori_loop(0, _NBLK, block_body, 0)


def kernel(x, perm):
    out = _permute_cols(x.reshape(-1), perm.astype(jnp.int32))
    return out.reshape(_ROWS, _DIM)


# speedup vs baseline: 1.9674x; 1.9674x over previous
import functools

import jax
import jax.numpy as jnp
from jax import lax
from jax.experimental import pallas as pl
from jax.experimental.pallas import tpu as pltpu
from jax.experimental.pallas import tpu_sc as plsc

_ROWS = 8192
_DIM = 4096
_L = 16
_NC = 2
_NS = 16
_NW = _NC * _NS
_RPT = _ROWS // _NW
_RB = 4
_BLK = _RB * _DIM
_NBLK = _RPT // _RB
_NPAIR = _NBLK // 2
_JC = _DIM // _L

_mesh = plsc.VectorSubcoreMesh(core_axis_name="c", subcore_axis_name="s")


@functools.partial(
    pl.kernel,
    out_type=jax.ShapeDtypeStruct((_ROWS * _DIM,), jnp.float32),
    mesh=_mesh,
    scratch_types=[
        pltpu.VMEM((_DIM,), jnp.int32),
        pltpu.VMEM((_BLK,), jnp.float32),
        pltpu.VMEM((_BLK,), jnp.float32),
        pltpu.VMEM((_BLK,), jnp.float32),
        pltpu.VMEM((_BLK,), jnp.float32),
        pltpu.SemaphoreType.DMA,
        pltpu.SemaphoreType.DMA,
        pltpu.SemaphoreType.DMA,
        pltpu.SemaphoreType.DMA,
    ],
    compiler_params=pltpu.CompilerParams(needs_layout_passes=False),
)
def _permute_cols(x_hbm, perm_hbm, out_hbm, perm_v, in0, in1, out0, out1,
                  isem0, isem1, osem0, osem1):
    wid = lax.axis_index("s") * _NC + lax.axis_index("c")
    tile_base = wid * _RPT * _DIM
    pltpu.sync_copy(perm_hbm, perm_v)

    def start_in(b, buf, sem):
        pltpu.async_copy(x_hbm.at[pl.ds(tile_base + b * _BLK, _BLK)], buf, sem)

    def wait_dma(buf, sem):
        pltpu.make_async_copy(x_hbm.at[pl.ds(tile_base, _BLK)], buf, sem).wait()

    def start_out(b, buf, sem):
        pltpu.async_copy(buf, out_hbm.at[pl.ds(tile_base + b * _BLK, _BLK)], sem)

    def gather_block(in_buf, out_buf):
        @plsc.parallel_loop(0, _JC, step=1, unroll=4)
        def _(jc):
            col = jc * _L
            idx = perm_v[pl.ds(col, _L)]
            for r in range(_RB):
                vals = plsc.load_gather(in_buf, [idx + r * _DIM])
                out_buf[pl.ds(r * _DIM + col, _L)] = vals

    start_in(0, in0, isem0)
    start_in(1, in1, isem1)

    def pair_body(g, carry):
        b0 = 2 * g

        wait_dma(in0, isem0)

        @pl.when(g > 0)
        def _():
            wait_dma(out0, osem0)

        gather_block(in0, out0)
        start_out(b0, out0, osem0)

        @pl.when(g < _NPAIR - 1)
        def _():
            start_in(b0 + 2, in0, isem0)

        wait_dma(in1, isem1)

        @pl.when(g > 0)
        def _():
            wait_dma(out1, osem1)

        gather_block(in1, out1)
        start_out(b0 + 1, out1, osem1)

        @pl.when(g < _NPAIR - 1)
        def _():
            start_in(b0 + 3, in1, isem1)

        return carry

    lax.fori_loop(0, _NPAIR, pair_body, 0)
    wait_dma(out0, osem0)
    wait_dma(out1, osem1)


def kernel(x, perm):
    out = _permute_cols(x.reshape(-1), perm.astype(jnp.int32))
    return out.reshape(_ROWS, _DIM)


# speedup vs baseline: 3.6438x; 1.8521x over previous
import functools

import jax
import jax.numpy as jnp
from jax import lax
from jax.experimental import pallas as pl
from jax.experimental.pallas import tpu as pltpu
from jax.experimental.pallas import tpu_sc as plsc

_ROWS = 8192
_DIM = 4096
_L = 16
_NC = 2
_NS = 16
_NW = _NC * _NS
_RPT = _ROWS // _NW
_RB = 8
_NSTR = _RPT // _RB
_JC = _DIM // _L

_mesh = plsc.VectorSubcoreMesh(core_axis_name="c", subcore_axis_name="s")


@functools.partial(
    pl.kernel,
    out_type=jax.ShapeDtypeStruct((_ROWS, _DIM), jnp.float32),
    mesh=_mesh,
    scratch_types=[
        pltpu.VMEM((_DIM,), jnp.int32),
        pltpu.VMEM((_RB, _DIM), jnp.float32),
        pltpu.VMEM((_RB, _DIM), jnp.float32),
    ],
    compiler_params=pltpu.CompilerParams(needs_layout_passes=False),
)
def _permute_cols(x_hbm, perm_hbm, out_hbm, perm_v, in_v, out_v):
    wid = lax.axis_index("s") * _NC + lax.axis_index("c")
    row_base = wid * _RPT
    pltpu.sync_copy(perm_hbm, perm_v)

    def stripe_body(s, carry):
        row0 = row_base + s * _RB
        pltpu.sync_copy(x_hbm.at[pl.ds(row0, _RB)], in_v)

        @plsc.parallel_loop(0, _JC, step=1, unroll=4)
        def _(jc):
            col = jc * _L
            idx = perm_v[pl.ds(col, _L)]
            for r in range(_RB):
                rv = jnp.full((_L,), r, dtype=jnp.int32)
                vals = plsc.load_gather(in_v, [rv, idx])
                out_v[r, pl.ds(col, _L)] = vals

        pltpu.sync_copy(out_v, out_hbm.at[pl.ds(row0, _RB)])
        return carry

    lax.fori_loop(0, _NSTR, stripe_body, 0)


def kernel(x, perm):
    return _permute_cols(x, perm.astype(jnp.int32))


# speedup vs baseline: 5.9073x; 1.6212x over previous
import functools

import jax
import jax.numpy as jnp
from jax import lax
from jax.experimental import pallas as pl
from jax.experimental.pallas import tpu as pltpu
from jax.experimental.pallas import tpu_sc as plsc

_ROWS = 8192
_DIM = 4096
_HALF = _DIM // 2
_L = 16
_NC = 2
_NS = 16
_NW = _NC * _NS
_RPT = _ROWS // _NW
_RB = 8
_NSTR = _RPT // _RB
_NPAIR = _NSTR // 2
_HC = _HALF // _L

_mesh = plsc.VectorSubcoreMesh(core_axis_name="c", subcore_axis_name="s")


@functools.partial(
    pl.kernel,
    out_type=jax.ShapeDtypeStruct((_ROWS, _DIM), jnp.float32),
    mesh=_mesh,
    scratch_types=[
        pltpu.VMEM((_DIM,), jnp.int32),
        pltpu.VMEM((_RB, _DIM), jnp.float32),
        pltpu.VMEM((_RB, _DIM), jnp.float32),
        pltpu.VMEM((_RB, _HALF), jnp.float32),
        pltpu.VMEM((_RB, _HALF), jnp.float32),
        pltpu.SemaphoreType.DMA,
        pltpu.SemaphoreType.DMA,
        pltpu.SemaphoreType.DMA,
        pltpu.SemaphoreType.DMA,
    ],
    compiler_params=pltpu.CompilerParams(needs_layout_passes=False),
)
def _permute_cols(x_hbm, perm_hbm, out_hbm, perm_v, in0, in1, outa, outb,
                  isem0, isem1, osema, osemb):
    wid = lax.axis_index("s") * _NC + lax.axis_index("c")
    row_base = wid * _RPT
    pltpu.sync_copy(perm_hbm, perm_v)

    def start_in(s, buf, sem):
        pltpu.async_copy(x_hbm.at[pl.ds(row_base + s * _RB, _RB)], buf, sem)

    def wait_in(buf, sem):
        pltpu.make_async_copy(x_hbm.at[pl.ds(row_base, _RB)], buf, sem).wait()

    def start_out(s, h, buf, sem):
        dst = out_hbm.at[pl.ds(row_base + s * _RB, _RB), pl.ds(h * _HALF, _HALF)]
        pltpu.async_copy(buf, dst, sem)

    def wait_out(buf, sem):
        src = x_hbm.at[pl.ds(row_base, _RB), pl.ds(0, _HALF)]
        pltpu.make_async_copy(src, buf, sem).wait()

    def gather_half(in_buf, out_buf, h):
        @plsc.parallel_loop(0, _HC, step=1, unroll=4)
        def _(jc):
            col = h * _HALF + jc * _L
            idx = perm_v[pl.ds(col, _L)]
            for r in range(_RB):
                rv = jnp.full((_L,), r, dtype=jnp.int32)
                vals = plsc.load_gather(in_buf, [rv, idx])
                out_buf[r, pl.ds(jc * _L, _L)] = vals

    start_in(0, in0, isem0)
    start_in(1, in1, isem1)

    def pair_body(p, carry):
        s0 = 2 * p

        wait_in(in0, isem0)

        @pl.when(p > 0)
        def _():
            wait_out(outa, osema)

        gather_half(in0, outa, 0)
        start_out(s0, 0, outa, osema)

        @pl.when(p > 0)
        def _():
            wait_out(outb, osemb)

        gather_half(in0, outb, 1)
        start_out(s0, 1, outb, osemb)

        @pl.when(p < _NPAIR - 1)
        def _():
            start_in(s0 + 2, in0, isem0)

        wait_in(in1, isem1)
        wait_out(outa, osema)
        gather_half(in1, outa, 0)
        start_out(s0 + 1, 0, outa, osema)

        wait_out(outb, osemb)
        gather_half(in1, outb, 1)
        start_out(s0 + 1, 1, outb, osemb)

        @pl.when(p < _NPAIR - 1)
        def _():
            start_in(s0 + 3, in1, isem1)

        return carry

    lax.fori_loop(0, _NPAIR, pair_body, 0)
    wait_out(outa, osema)
    wait_out(outb, osemb)


def kernel(x, perm):
    return _permute_cols(x, perm.astype(jnp.int32))
